# baseline (speedup 1.0000x reference)
import jax
import jax.numpy as jnp
from jax.experimental import pallas as pl
from jax.experimental.pallas import tpu as pltpu

_B, _H, _W, _C = 16, 256, 256, 64
_HH = _H // 2
_WH = _W // 2


def _cross_shift_kernel(x_hbm, o_hbm, zrow, zcol, quad_sems, zrow_sem, zcol_sem):
    zrow[...] = jnp.zeros_like(zrow)
    zcol[...] = jnp.zeros_like(zcol)

    for b in range(_B):
        xb = x_hbm.at[b]
        ob = o_hbm.at[b]
        pltpu.make_async_copy(
            xb.at[0:_HH, 0:_WH],
            ob.at[0:_HH, 0:_WH],
            quad_sems.at[0]).start()
        pltpu.make_async_copy(
            xb.at[0:_HH, _WH:_W],
            ob.at[0:_HH, _WH + 1:_W + 1],
            quad_sems.at[1]).start()
        pltpu.make_async_copy(
            xb.at[_HH:_H, 0:_WH],
            ob.at[_HH + 1:_H + 1, 0:_WH],
            quad_sems.at[2]).start()
        pltpu.make_async_copy(
            xb.at[_HH:_H, _WH:_W],
            ob.at[_HH + 1:_H + 1, _WH + 1:_W + 1],
            quad_sems.at[3]).start()
        pltpu.make_async_copy(
            zrow, ob.at[_HH:_HH + 1, :], zrow_sem).start()
        pltpu.make_async_copy(
            zcol, ob.at[0:_HH, _WH:_WH + 1], zcol_sem).start()
        pltpu.make_async_copy(
            zcol, ob.at[_HH + 1:_H + 1, _WH:_WH + 1], zcol_sem).start()

    for q in range(4):
        pltpu.make_async_copy(
            x_hbm.at[:, 0:_HH, 0:_WH],
            x_hbm.at[:, 0:_HH, 0:_WH],
            quad_sems.at[q]).wait()
    pltpu.make_async_copy(
        o_hbm.at[:, _HH:_HH + 1, :],
        o_hbm.at[:, _HH:_HH + 1, :],
        zrow_sem).wait()
    pltpu.make_async_copy(
        x_hbm.at[:, 0:_H, _WH:_WH + 1],
        x_hbm.at[:, 0:_H, _WH:_WH + 1],
        zcol_sem).wait()


def kernel(x):
    x5 = x.reshape(_B, _H, _W, 1, _C)
    out = pl.pallas_call(
        _cross_shift_kernel,
        out_shape=jax.ShapeDtypeStruct((_B, _H + 1, _W + 1, 1, _C), x.dtype),
        in_specs=[pl.BlockSpec(memory_space=pl.ANY)],
        out_specs=pl.BlockSpec(memory_space=pl.ANY),
        scratch_shapes=[
            pltpu.VMEM((1, _W + 1, 1, _C), jnp.float32),
            pltpu.VMEM((_HH, 1, 1, _C), jnp.float32),
            pltpu.SemaphoreType.DMA((4,)),
            pltpu.SemaphoreType.DMA,
            pltpu.SemaphoreType.DMA,
        ],
        name="cross_shift",
    )(x5)
    return out.reshape(_B, _H + 1, _W + 1, _C)


# speedup vs baseline: 15.2846x; 15.2846x over previous
import jax
import jax.numpy as jnp
from jax.experimental import pallas as pl
from jax.experimental.pallas import tpu as pltpu

_B, _H, _W, _C = 16, 256, 256, 64
_HB = _H // 2
_WB = _W // 2


def _cross_shift_kernel(x_ref, o_ref):
    jh = pl.program_id(1)
    jw = pl.program_id(2)
    v = x_ref[...]

    @pl.when(jw == 0)
    def _():
        o_ref[:, 0:_HB] = v

    @pl.when(jw == 1)
    def _():
        shifted = jnp.concatenate(
            [jnp.zeros((1, _HB, 1, _C), v.dtype), v[:, :, 0:_WB - 1, :]],
            axis=2)
        o_ref[:, 0:_HB] = shifted

    @pl.when(jw == 2)
    def _():
        o_ref[:, 0:_HB, 0:1] = v[:, :, _WB - 1:_WB, :]

    @pl.when(jh == 0)
    def _():
        o_ref[:, _HB:_HB + 1] = jnp.zeros_like(o_ref[:, _HB:_HB + 1])


def kernel(x):
    return pl.pallas_call(
        _cross_shift_kernel,
        out_shape=jax.ShapeDtypeStruct((_B, _H + 1, _W + 1, _C), x.dtype),
        grid=(_B, 2, 3),
        in_specs=[pl.BlockSpec(
            (1, _HB, _WB, _C),
            lambda b, jh, jw: (b, jh, jnp.minimum(jw, 1), 0))],
        out_specs=pl.BlockSpec(
            (1, _HB + 1, _WB, _C),
            lambda b, jh, jw: (b, jh, jw, 0)),
        compiler_params=pltpu.CompilerParams(
            dimension_semantics=("parallel", "arbitrary", "arbitrary"),
            vmem_limit_bytes=48 * 1024 * 1024,
        ),
        name="cross_shift",
    )(x)


# speedup vs baseline: 15.4618x; 1.0116x over previous
import jax
import jax.numpy as jnp
from jax.experimental import pallas as pl
from jax.experimental.pallas import tpu as pltpu

_B, _H, _W, _C = 16, 256, 256, 64
_RB = 64
_CPB = _H // _RB
_NSTEP = _B * _CPB
_NSLOT = 3


def _cross_shift_kernel(x_hbm, o_hbm, in_bufs, out_bufs, zrow,
                        in_sems, out_sems, zrow_sem):
    s = pl.program_id(0)
    b = s // _CPB
    c = s % _CPB

    def in_copy(step):
        bb = step // _CPB
        cc = step % _CPB
        return pltpu.make_async_copy(
            x_hbm.at[bb, pl.ds(cc * _RB, _RB)],
            in_bufs.at[step % _NSLOT],
            in_sems.at[step % _NSLOT])

    def out_copy(step):
        bb = step // _CPB
        cc = step % _CPB
        off = cc * _RB + jnp.where(cc >= _CPB // 2, 1, 0)
        return pltpu.make_async_copy(
            out_bufs.at[step % _NSLOT],
            o_hbm.at[bb, pl.ds(off, _RB)],
            out_sems.at[step % _NSLOT])

    @pl.when(s == 0)
    def _():
        zrow[...] = jnp.zeros_like(zrow)
        in_copy(0).start()

    @pl.when(s + 1 < _NSTEP)
    def _():
        in_copy(s + 1).start()

    in_copy(s).wait()

    @pl.when(s >= _NSLOT)
    def _():
        out_copy(s - _NSLOT).wait()

    slot = s % _NSLOT
    for k in range(_NSLOT):
        @pl.when(slot == k)
        def _(k=k):
            v = in_bufs[k]
            out_bufs[k] = jnp.concatenate(
                [v[:, 0:_W // 2],
                 jnp.zeros((_RB, 1, _C), v.dtype),
                 v[:, _W // 2:]],
                axis=1)

    @pl.when(c == _CPB // 2)
    def _():
        pltpu.make_async_copy(
            zrow, o_hbm.at[b, pl.ds(_H // 2, 1)], zrow_sem).start()

    out_copy(s).start()

    @pl.when(s == _NSTEP - 1)
    def _():
        for k in range(_NSLOT):
            out_copy(_NSTEP - _NSLOT + k).wait()
        pltpu.make_async_copy(
            o_hbm.at[:, _H // 2:_H // 2 + 1],
            o_hbm.at[:, _H // 2:_H // 2 + 1],
            zrow_sem).wait()


def kernel(x):
    return pl.pallas_call(
        _cross_shift_kernel,
        out_shape=jax.ShapeDtypeStruct((_B, _H + 1, _W + 1, _C), x.dtype),
        grid=(_NSTEP,),
        in_specs=[pl.BlockSpec(memory_space=pl.ANY)],
        out_specs=pl.BlockSpec(memory_space=pl.ANY),
        scratch_shapes=[
            pltpu.VMEM((_NSLOT, _RB, _W, _C), jnp.float32),
            pltpu.VMEM((_NSLOT, _RB, _W + 1, _C), jnp.float32),
            pltpu.VMEM((1, _W + 1, _C), jnp.float32),
            pltpu.SemaphoreType.DMA((_NSLOT,)),
            pltpu.SemaphoreType.DMA((_NSLOT,)),
            pltpu.SemaphoreType.DMA,
        ],
        compiler_params=pltpu.CompilerParams(
            dimension_semantics=("arbitrary",),
            vmem_limit_bytes=56 * 1024 * 1024,
        ),
        name="cross_shift",
    )(x)
